# Initial kernel scaffold; baseline (speedup 1.0000x reference)
import jax
import jax.numpy as jnp
from jax.experimental import pallas as pl


def kernel(features, W0, b0, W1, b1, W2, b2, edge_index_0, edge_weight_0, edge_index_1, edge_weight_1, edge_index_2, edge_weight_2):
    raise NotImplementedError("write your pallas kernel here")


# speedup vs baseline: 3.7907x; 3.7907x over previous
import dataclasses
import functools

import jax
import jax.numpy as jnp
from jax import lax
from jax.experimental import pallas as pl
from jax.experimental.pallas import tpu as pltpu
from jax.experimental.pallas import tpu_sc as plsc

N = 10000
D = 128
E = 320000
V = 3
LAMBD = 0.01

NC = 2
NS = 16
NW = NC * NS
LANES = 16
CHUNK = 128
NCHUNK = E // CHUNK
TRIPS = (NCHUNK + NW - 1) // NW
ROWS_PER_TILE = 624
TAIL_BASE = ROWS_PER_TILE * NS
TAIL_ROWS = N - TAIL_BASE


_ENC_BLK = 2000


def _encode_body(f_ref, w_ref, b_ref, o0_ref, o1_ref, o2_ref):
    f = f_ref[...]
    outs = (o0_ref, o1_ref, o2_ref)
    for i in range(V):
        h = lax.dot_general(
            f, w_ref[i],
            dimension_numbers=(((1,), (0,)), ((), ())),
            preferred_element_type=jnp.float32,
            precision=lax.Precision.HIGHEST,
        )
        outs[i][...] = h + b_ref[i][None, :]


def _encode(features, w_stack, b_stack):
    nblk = N // _ENC_BLK
    return pl.pallas_call(
        _encode_body,
        grid=(nblk,),
        in_specs=[
            pl.BlockSpec((_ENC_BLK, D), lambda ib: (ib, 0)),
            pl.BlockSpec((V, D, D), lambda ib: (0, 0, 0)),
            pl.BlockSpec((V, D), lambda ib: (0, 0)),
        ],
        out_specs=[pl.BlockSpec((_ENC_BLK, D), lambda ib: (ib, 0))] * V,
        out_shape=[jax.ShapeDtypeStruct((N, D), jnp.float32)] * V,
    )(features, w_stack, b_stack)


def _spmm_body(ha0, ha1, ha2, src0, dst0, w0, src1, dst1, w1, src2, dst2, w2,
               out_ref, six, dix, wv, rows, hp_sh):
    cid = lax.axis_index("c")
    sid = lax.axis_index("s")
    wid = sid * NC + cid
    base = sid * ROWS_PER_TILE

    has = (ha0, ha1, ha2)
    srcs = (src0, src1, src2)
    dsts = (dst0, dst1, dst2)
    ws = (w0, w1, w2)

    for i in range(V):
        @pl.loop(0, CHUNK)
        def _(j):
            for cc in range(D // LANES):
                rows[j, pl.ds(cc * LANES, LANES)] = jnp.zeros(
                    (LANES,), jnp.float32)

        for k in range(ROWS_PER_TILE // CHUNK):
            pltpu.sync_copy(rows, hp_sh.at[pl.ds(base + k * CHUNK, CHUNK)])
        rem = ROWS_PER_TILE % CHUNK
        if rem:
            pltpu.sync_copy(
                rows.at[pl.ds(0, rem)],
                hp_sh.at[pl.ds(base + (ROWS_PER_TILE // CHUNK) * CHUNK, rem)])

        @pl.when(sid == 0)
        def _():
            pltpu.sync_copy(rows.at[pl.ds(0, TAIL_ROWS)],
                            hp_sh.at[pl.ds(TAIL_BASE, TAIL_ROWS)])

        plsc.subcore_barrier()

        @pl.loop(0, TRIPS)
        def _(t):
            c = wid + t * NW

            @pl.when(c < NCHUNK)
            def _():
                eb = c * CHUNK
                pltpu.sync_copy(srcs[i].at[pl.ds(eb, CHUNK)], six)
                pltpu.sync_copy(dsts[i].at[pl.ds(eb, CHUNK)], dix)
                pltpu.sync_copy(ws[i].at[pl.ds(eb, CHUNK)], wv)
                pltpu.sync_copy(has[i].at[six], rows)

                @pl.loop(0, CHUNK)
                def _(j):
                    wj = plsc.load_gather(
                        wv, [jnp.full((LANES,), j, jnp.int32)])
                    for cc in range(D // LANES):
                        sl = pl.ds(cc * LANES, LANES)
                        rows[j, sl] = rows[j, sl] * wj

                pltpu.sync_copy(rows, hp_sh.at[dix], add=True)

        plsc.subcore_barrier()

        for k in range(ROWS_PER_TILE // CHUNK):
            pltpu.sync_copy(
                hp_sh.at[pl.ds(base + k * CHUNK, CHUNK)],
                out_ref.at[i, cid, pl.ds(base + k * CHUNK, CHUNK)])
        if rem:
            pltpu.sync_copy(
                hp_sh.at[pl.ds(base + (ROWS_PER_TILE // CHUNK) * CHUNK, rem)],
                out_ref.at[i, cid,
                           pl.ds(base + (ROWS_PER_TILE // CHUNK) * CHUNK, rem)])

        @pl.when(sid == 0)
        def _():
            pltpu.sync_copy(hp_sh.at[pl.ds(TAIL_BASE, TAIL_ROWS)],
                            out_ref.at[i, cid, pl.ds(TAIL_BASE, TAIL_ROWS)])


def _spmm(ha0, ha1, ha2, edges):
    mesh = plsc.VectorSubcoreMesh(core_axis_name="c", subcore_axis_name="s")
    cp = pltpu.CompilerParams()
    if "needs_layout_passes" in pltpu.CompilerParams.__dataclass_fields__:
        cp = dataclasses.replace(cp, needs_layout_passes=False)
    k = pl.kernel(
        _spmm_body,
        out_type=jax.ShapeDtypeStruct((V, NC, N, D), jnp.float32),
        mesh=mesh,
        compiler_params=cp,
        scratch_types=[
            pltpu.VMEM((CHUNK,), jnp.int32),
            pltpu.VMEM((CHUNK,), jnp.int32),
            pltpu.VMEM((CHUNK,), jnp.float32),
            pltpu.VMEM((CHUNK, D), jnp.float32),
            pltpu.VMEM_SHARED((N, D), jnp.float32),
        ],
    )
    args = []
    args.extend([ha0, ha1, ha2])
    for (src, dst, w) in edges:
        args.extend([src, dst, w])
    return k(args[0], args[1], args[2],
             args[3], args[4], args[5],
             args[6], args[7], args[8],
             args[9], args[10], args[11])


_COR_BLK = 1000


def _corr_body(ha0, ha1, ha2, hp_ref, o_ref, acc):
    ib = pl.program_id(0)
    nblk = pl.num_programs(0)
    has = (ha0[...], ha1[...], ha2[...])
    hps = tuple(hp_ref[i, 0] + hp_ref[i, 1] for i in range(V))

    @pl.when(ib == 0)
    def _():
        acc[...] = jnp.zeros_like(acc)

    def _xtx(a, b):
        return lax.dot_general(
            a, b,
            dimension_numbers=(((0,), (0,)), ((), ())),
            preferred_element_type=jnp.float32,
            precision=lax.Precision.HIGHEST,
        )

    for i in range(V):
        acc[i] += _xtx(hps[i], has[i])
        acc[V + i] += _xtx(hps[i], hps[(i + 1) % V])

    @pl.when(ib == nblk - 1)
    def _():
        r = lax.broadcasted_iota(jnp.int32, (D, D), 0)
        c = lax.broadcasted_iota(jnp.int32, (D, D), 1)
        eye = (r == c).astype(jnp.float32)
        loss = jnp.float32(0.0)
        for j in range(2 * V):
            m = acc[j]
            on_diag = jnp.sum(eye * (m - 1.0) ** 2)
            off_diag = jnp.sum(m * m) - jnp.sum(eye * m * m)
            loss = loss + on_diag + LAMBD * off_diag
        o_ref[...] = jnp.full((1, 1), loss, jnp.float32)


def _corr(ha0, ha1, ha2, hp):
    nblk = N // _COR_BLK
    return pl.pallas_call(
        _corr_body,
        grid=(nblk,),
        in_specs=[
            pl.BlockSpec((_COR_BLK, D), lambda ib: (ib, 0)),
            pl.BlockSpec((_COR_BLK, D), lambda ib: (ib, 0)),
            pl.BlockSpec((_COR_BLK, D), lambda ib: (ib, 0)),
            pl.BlockSpec((V, NC, _COR_BLK, D), lambda ib: (0, 0, ib, 0)),
        ],
        out_specs=pl.BlockSpec((1, 1), lambda ib: (0, 0)),
        out_shape=jax.ShapeDtypeStruct((1, 1), jnp.float32),
        scratch_shapes=[pltpu.VMEM((2 * V, D, D), jnp.float32)],
    )(ha0, ha1, ha2, hp)


def kernel(features, W0, b0, W1, b1, W2, b2, edge_index_0, edge_weight_0,
           edge_index_1, edge_weight_1, edge_index_2, edge_weight_2):
    w_stack = jnp.stack([W0, W1, W2])
    b_stack = jnp.stack([b0, b1, b2])
    ha0, ha1, ha2 = _encode(features, w_stack, b_stack)
    edges = [
        (edge_index_0[1], edge_index_0[0], edge_weight_0),
        (edge_index_1[1], edge_index_1[0], edge_weight_1),
        (edge_index_2[1], edge_index_2[0], edge_weight_2),
    ]
    hp = _spmm(ha0, ha1, ha2, edges)
    loss = _corr(ha0, ha1, ha2, hp)
    return loss[0, 0]
